# Initial kernel scaffold; baseline (speedup 1.0000x reference)
import jax
import jax.numpy as jnp
from jax.experimental import pallas as pl


def kernel(input_ids, token_emb, fc_w, fc_b):
    raise NotImplementedError("write your pallas kernel here")


# speedup vs baseline: 3.1089x; 3.1089x over previous
import functools

import jax
import jax.numpy as jnp
from jax import lax
from jax.experimental import pallas as pl
from jax.experimental.pallas import tpu as pltpu
from jax.experimental.pallas import tpu_sc as plsc

PADW = 16


def _proj_body(emb_ref, w_ref, b_ref, out_ref):
    out_ref[...] = (
        jnp.dot(emb_ref[...], w_ref[...], preferred_element_type=jnp.float32)
        + b_ref[...]
    )


@functools.partial(jax.jit, static_argnames=("blk",))
def _project(token_emb, w_pad, b_pad, blk=2048):
    vocab, embed = token_emb.shape
    grid = (pl.cdiv(vocab, blk),)
    return pl.pallas_call(
        _proj_body,
        grid=grid,
        in_specs=[
            pl.BlockSpec((blk, embed), lambda i: (i, 0)),
            pl.BlockSpec((embed, PADW), lambda i: (0, 0)),
            pl.BlockSpec((1, PADW), lambda i: (0, 0)),
        ],
        out_specs=pl.BlockSpec((blk, PADW), lambda i: (i, 0)),
        out_shape=jax.ShapeDtypeStruct((vocab, PADW), jnp.float32),
    )(token_emb, w_pad, b_pad)


@functools.cache
def _make_gather(batch):
    info = plsc.get_sparse_core_info()
    nc, ns = info.num_cores, info.num_subcores
    nw = nc * ns
    assert batch % (8 * nw) == 0
    b_per_w = batch // nw
    mesh = plsc.VectorSubcoreMesh(core_axis_name="c", subcore_axis_name="s")

    @functools.partial(
        pl.kernel,
        mesh=mesh,
        compiler_params=pltpu.CompilerParams(use_tc_tiling_on_sc=False),
        out_type=jax.ShapeDtypeStruct((batch, PADW), jnp.float32),
        scratch_types=[
            pltpu.VMEM((b_per_w,), jnp.int32),
            pltpu.VMEM((b_per_w, PADW), jnp.float32),
            pltpu.SemaphoreType.DMA,
        ],
    )
    def gather(table_hbm, idx_hbm, out_hbm, idx_v, rows_v, sem):
        wid = lax.axis_index("s") * nc + lax.axis_index("c")
        base = wid * b_per_w
        pltpu.sync_copy(idx_hbm.at[pl.ds(base, b_per_w)], idx_v)
        pltpu.async_copy(table_hbm.at[idx_v], rows_v, sem).wait()
        pltpu.sync_copy(rows_v, out_hbm.at[pl.ds(base, b_per_w)])

    return gather


def kernel(input_ids, token_emb, fc_w, fc_b):
    batch = input_ids.shape[0]
    ids = input_ids[:, 0].astype(jnp.int32)
    w_pad = jnp.zeros((token_emb.shape[1], PADW), jnp.float32).at[:, :2].set(fc_w)
    b_pad = jnp.zeros((1, PADW), jnp.float32).at[0, :2].set(fc_b)
    proj = _project(token_emb, w_pad, b_pad)
    out16 = _make_gather(batch)(proj, ids)
    return out16[:, :2]
